# baseline (speedup 1.0000x reference)
import functools

import jax
import jax.numpy as jnp
from jax import lax
from jax.experimental import pallas as pl
from jax.experimental.pallas import tpu as pltpu
from jax.experimental.pallas import tpu_sc as plsc

_INFO = plsc.get_sparse_core_info()
_NC = _INFO.num_cores
_NS = _INFO.num_subcores
_NW = _NC * _NS
_LANES = _INFO.num_lanes
_G = 2


@functools.lru_cache(maxsize=None)
def _make_embedding_bag(B, H, D, V):
    assert B % (_NW * _G) == 0
    S = B // (_NW * _G)
    R = _G * H
    RF = -(-R // 16) * 16
    P = 128
    assert RF <= P
    NCH = -(-RF // _LANES)
    BPW = B // _NW
    KD = D // _LANES

    mesh = plsc.VectorSubcoreMesh(core_axis_name="c", subcore_axis_name="s")

    @functools.partial(
        pl.kernel,
        mesh=mesh,
        compiler_params=pltpu.CompilerParams(use_tc_tiling_on_sc=True),
        out_type=jax.ShapeDtypeStruct((B // 2, 2 * D), jnp.float32),
        scratch_types=[
            pltpu.VMEM((S, P), jnp.int32),
            pltpu.VMEM((S, P), jnp.float32),
            [pltpu.VMEM((RF // 2, 2 * D), jnp.float32)] * 2,
            pltpu.VMEM((BPW // 2, 2 * D), jnp.float32),
            [pltpu.SemaphoreType.DMA] * 2,
        ],
    )
    def bag(table_hbm, idx_hbm, wgt_hbm, out_hbm,
            idx_v, wgt_v, rows_bufs, out_v, sems):
        wid = lax.axis_index("s") * _NC + lax.axis_index("c")
        pltpu.sync_copy(idx_hbm.at[wid], idx_v)
        pltpu.sync_copy(wgt_hbm.at[wid], wgt_v)

        def fire(s, buf, sem):
            ivecs = [idx_v[s, pl.ds(c * _LANES, _LANES)] for c in range(NCH)]
            for p in range(RF):
                r = ivecs[p // _LANES][p % _LANES]
                pltpu.async_copy(
                    table_hbm.at[r], buf.at[p // 2, pl.ds((p % 2) * D, D)], sem)

        def drain(buf, sem):
            pltpu.make_async_copy(
                out_hbm.at[pl.ds(0, RF // 2)], buf, sem).wait()

        def phase(s, cur_buf, cur_sem, nxt_buf, nxt_sem):
            snxt = jnp.where(s + 1 < S, s + 1, 0)
            drain(cur_buf, cur_sem)
            ivecs = [idx_v[snxt, pl.ds(c * _LANES, _LANES)]
                     for c in range(NCH)]
            wvecs = [wgt_v[s, pl.ds(c * _LANES, _LANES)] for c in range(NCH)]
            accs = [[jnp.zeros((_LANES,), jnp.float32) for _ in range(KD)]
                    for _ in range(_G)]
            for p in range(RF):
                r = ivecs[p // _LANES][p % _LANES]
                pltpu.async_copy(
                    table_hbm.at[r],
                    nxt_buf.at[p // 2, pl.ds((p % 2) * D, D)], nxt_sem)
                if p >= R:
                    continue
                j = p // H
                w = wvecs[p // _LANES][p % _LANES]
                for k in range(KD):
                    off = (p % 2) * D + k * _LANES
                    accs[j][k] = (accs[j][k]
                                  + cur_buf[p // 2, pl.ds(off, _LANES)] * w)
            for j in range(_G):
                for k in range(KD):
                    out_v[s, pl.ds(j * D + k * _LANES, _LANES)] = accs[j][k]

        fire(0, rows_bufs[0], sems[0])

        def round_(t, carry):
            phase(2 * t, rows_bufs[0], sems[0], rows_bufs[1], sems[1])
            phase(2 * t + 1, rows_bufs[1], sems[1], rows_bufs[0], sems[0])
            return carry

        lax.fori_loop(0, S // 2, round_, 0)
        drain(rows_bufs[0], sems[0])
        pltpu.sync_copy(out_v, out_hbm.at[pl.ds(wid * (BPW // 2), BPW // 2)])

    return bag


def kernel(X, X_w, W):
    B, H = X.shape
    V, D = W.shape
    S = B // (_NW * _G)
    R = _G * H
    Xr = X.astype(jnp.int32).reshape(_NW, S, R)
    Wr = X_w.astype(jnp.float32).reshape(_NW, S, R)
    Xp = jnp.pad(Xr, ((0, 0), (0, 0), (0, 128 - R)))
    Wp = jnp.pad(Wr, ((0, 0), (0, 0), (0, 128 - R)))
    out2 = _make_embedding_bag(B, H, D, V)(W, Xp, Wp)
    return out2.reshape(B, D)


# speedup vs baseline: 1.1828x; 1.1828x over previous
import functools

import jax
import jax.numpy as jnp
from jax import lax
from jax.experimental import pallas as pl
from jax.experimental.pallas import tpu as pltpu
from jax.experimental.pallas import tpu_sc as plsc

_INFO = plsc.get_sparse_core_info()
_NC = _INFO.num_cores
_NS = _INFO.num_subcores
_NW = _NC * _NS
_LANES = _INFO.num_lanes
_G = 2
_PD = 128


@functools.lru_cache(maxsize=None)
def _make_embedding_bag(B, H, D, V):
    assert B % (_NW * _G) == 0
    S = B // (_NW * _G)
    R = _G * H
    RF = -(-R // 8) * 8
    BPW = B // _NW
    KD = D // _LANES

    mesh = plsc.VectorSubcoreMesh(core_axis_name="c", subcore_axis_name="s")

    @functools.partial(
        pl.kernel,
        mesh=mesh,
        compiler_params=pltpu.CompilerParams(use_tc_tiling_on_sc=True),
        out_type=jax.ShapeDtypeStruct((B // _G, _G * D), jnp.float32),
        scratch_types=[
            pltpu.VMEM((S, _PD), jnp.int32),
            pltpu.VMEM((S, _PD), jnp.float32),
            [pltpu.VMEM((RF, _PD), jnp.float32)] * 2,
            pltpu.VMEM((S, _G * D), jnp.float32),
            [pltpu.SemaphoreType.DMA] * 2,
        ],
    )
    def bag(table_hbm, idx_hbm, wgt_hbm, out_hbm,
            idx_v, wgt_v, rows_bufs, out_v, sems):
        wid = lax.axis_index("s") * _NC + lax.axis_index("c")
        pltpu.sync_copy(idx_hbm.at[wid], idx_v)
        pltpu.sync_copy(wgt_hbm.at[wid], wgt_v)

        def gcopy(s, b):
            return pltpu.make_async_copy(
                table_hbm.at[idx_v.at[s, pl.ds(0, RF)]],
                rows_bufs[b], sems[b])

        def compute(s, rows_v):
            for j in range(_G):
                base = j * H
                chunk_offs = [0, 16, 32, H - _LANES]
                wvecs = [wgt_v[s, pl.ds(base + o, _LANES)] for o in chunk_offs]

                def wlane(n):
                    if n < 48:
                        return wvecs[n // 16][n % 16]
                    return wvecs[3][n - (H - _LANES)]

                acc = [rows_v[j * H, pl.ds(k * _LANES, _LANES)] * wlane(0)
                       for k in range(KD)]
                for n in range(1, H):
                    p = j * H + n
                    w = wlane(n)
                    for k in range(KD):
                        acc[k] = acc[k] + rows_v[p, pl.ds(k * _LANES, _LANES)] * w
                for k in range(KD):
                    out_v[s, pl.ds(j * D + k * _LANES, _LANES)] = acc[k]

        gcopy(0, 0).start()

        def phase(s, b):
            nb = 1 - b

            @pl.when(s + 1 < S)
            def _():
                gcopy(s + 1, nb).start()

            gcopy(s, b).wait()
            compute(s, rows_bufs[b])

        def round_(t, carry):
            phase(2 * t, 0)
            phase(2 * t + 1, 1)
            return carry

        lax.fori_loop(0, S // 2, round_, 0)
        pltpu.sync_copy(out_v, out_hbm.at[pl.ds(wid * S, S)])

    return bag


def kernel(X, X_w, W):
    B, H = X.shape
    V, D = W.shape
    S = B // (_NW * _G)
    R = _G * H
    RF = -(-R // 8) * 8
    Wp = jnp.pad(W[:V - 1], ((0, 0), (0, _PD - D)))
    Xr = X.astype(jnp.int32).reshape(_NW, S, R)
    Xf = jnp.pad(Xr, ((0, 0), (0, 0), (0, _PD - R)))
    Wr = X_w.astype(jnp.float32).reshape(_NW, S, R)
    Wf = jnp.pad(Wr, ((0, 0), (0, 0), (0, _PD - R)))
    out2 = _make_embedding_bag(B, H, D, V)(Wp, Xf, Wf)
    return out2.reshape(B, D)


# speedup vs baseline: 1.7327x; 1.4649x over previous
import functools

import jax
import jax.numpy as jnp
from jax import lax
from jax.experimental import pallas as pl
from jax.experimental.pallas import tpu as pltpu
from jax.experimental.pallas import tpu_sc as plsc

_INFO = plsc.get_sparse_core_info()
_NC = _INFO.num_cores
_NS = _INFO.num_subcores
_NW = _NC * _NS
_LANES = _INFO.num_lanes
_G = 2
_PR = 128
_NBUF = 4


@functools.lru_cache(maxsize=None)
def _make_embedding_bag(B, H, D, V):
    assert B % (_NW * _G) == 0
    S = B // (_NW * _G)
    R = _G * H
    RF = -(-R // 8) * 8
    assert RF <= _PR
    KD = D // _LANES

    mesh = plsc.VectorSubcoreMesh(core_axis_name="c", subcore_axis_name="s")

    @functools.partial(
        pl.kernel,
        mesh=mesh,
        compiler_params=pltpu.CompilerParams(use_tc_tiling_on_sc=False),
        out_type=jax.ShapeDtypeStruct((B // _G, _G * D), jnp.float32),
        scratch_types=[
            pltpu.VMEM((S, _PR), jnp.int32),
            pltpu.VMEM((S, _PR), jnp.float32),
            [pltpu.VMEM((RF, D), jnp.float32)] * _NBUF,
            pltpu.VMEM((S, _G * D), jnp.float32),
            [pltpu.SemaphoreType.DMA] * _NBUF,
        ],
    )
    def bag(table_hbm, idx_hbm, wgt_hbm, out_hbm,
            idx_v, wgt_v, rows_bufs, out_v, sems):
        wid = lax.axis_index("s") * _NC + lax.axis_index("c")
        pltpu.sync_copy(idx_hbm.at[wid], idx_v)
        pltpu.sync_copy(wgt_hbm.at[wid], wgt_v)

        def gcopy(s, b):
            return pltpu.make_async_copy(
                table_hbm.at[idx_v.at[s, pl.ds(0, RF)]],
                rows_bufs[b], sems[b])

        def compute(s, rows_v):
            for j in range(_G):
                base = j * H
                chunk_offs = [0, 16, 32, H - _LANES]
                wvecs = [wgt_v[s, pl.ds(base + o, _LANES)] for o in chunk_offs]

                def wlane(n):
                    if n < 48:
                        return wvecs[n // 16][n % 16]
                    return wvecs[3][n - (H - _LANES)]

                acc = [rows_v[j * H, pl.ds(k * _LANES, _LANES)] * wlane(0)
                       for k in range(KD)]
                for n in range(1, H):
                    p = j * H + n
                    w = wlane(n)
                    for k in range(KD):
                        acc[k] = acc[k] + rows_v[p, pl.ds(k * _LANES, _LANES)] * w
                for k in range(KD):
                    out_v[s, pl.ds(j * D + k * _LANES, _LANES)] = acc[k]

        for i in range(_NBUF - 1):
            gcopy(i, i).start()

        def round_(t, carry):
            s0 = t * _NBUF
            for b in range(_NBUF):
                s = s0 + b
                gcopy(s, b).wait()
                compute(s, rows_bufs[b])
                nxt = s + _NBUF - 1

                @pl.when(nxt < S)
                def _():
                    gcopy(nxt, (b + _NBUF - 1) % _NBUF).start()
            return carry

        lax.fori_loop(0, S // _NBUF, round_, 0)
        pltpu.sync_copy(out_v, out_hbm.at[pl.ds(wid * S, S)])

    return bag


def kernel(X, X_w, W):
    B, H = X.shape
    V, D = W.shape
    S = B // (_NW * _G)
    R = _G * H
    Xr = X.astype(jnp.int32).reshape(_NW, S, R)
    Xp = jnp.pad(Xr, ((0, 0), (0, 0), (0, _PR - R)))
    Wr = X_w.astype(jnp.float32).reshape(_NW, S, R)
    Wp = jnp.pad(Wr, ((0, 0), (0, 0), (0, _PR - R)))
    out2 = _make_embedding_bag(B, H, D, V)(W, Xp, Wp)
    return out2.reshape(B, D)


# speedup vs baseline: 2.4810x; 1.4318x over previous
import functools

import jax
import jax.numpy as jnp
from jax import lax
from jax.experimental import pallas as pl
from jax.experimental.pallas import tpu as pltpu
from jax.experimental.pallas import tpu_sc as plsc

_INFO = plsc.get_sparse_core_info()
_NC = _INFO.num_cores
_NS = _INFO.num_subcores
_NW = _NC * _NS
_LANES = _INFO.num_lanes
_G = 2
_PR = 128
_PD = 128
_NBUF = 2


@functools.lru_cache(maxsize=None)
def _make_embedding_bag(B, H, D, V):
    assert B % (_NW * _G) == 0
    S = B // (_NW * _G)
    R = _G * H
    RF = -(-R // 8) * 8
    assert RF <= _PR
    KD = D // _LANES

    mesh = plsc.VectorSubcoreMesh(core_axis_name="c", subcore_axis_name="s")

    @functools.partial(
        pl.kernel,
        mesh=mesh,
        compiler_params=pltpu.CompilerParams(use_tc_tiling_on_sc=False),
        out_type=jax.ShapeDtypeStruct((B // _G, _G * D), jnp.float32),
        scratch_types=[
            pltpu.VMEM((S, _PR), jnp.int32),
            pltpu.VMEM((S, _PR), jnp.float32),
            [pltpu.VMEM((RF, _PD), jnp.float32)] * _NBUF,
            pltpu.VMEM((S, _G * D), jnp.float32),
            [pltpu.SemaphoreType.DMA] * _NBUF,
        ],
    )
    def bag(table_hbm, idx_hbm, wgt_hbm, out_hbm,
            idx_v, wgt_v, rows_bufs, out_v, sems):
        wid = lax.axis_index("s") * _NC + lax.axis_index("c")
        pltpu.sync_copy(idx_hbm.at[wid], idx_v)
        pltpu.sync_copy(wgt_hbm.at[wid], wgt_v)

        def gcopy(s, b):
            return pltpu.make_async_copy(
                table_hbm.at[idx_v.at[s, pl.ds(0, RF)]],
                rows_bufs[b], sems[b])

        def compute(s, rows_v):
            for j in range(_G):
                base = j * H
                chunk_offs = [0, 16, 32, H - _LANES]
                wvecs = [wgt_v[s, pl.ds(base + o, _LANES)] for o in chunk_offs]

                def wlane(n):
                    if n < 48:
                        return wvecs[n // 16][n % 16]
                    return wvecs[3][n - (H - _LANES)]

                acc = [rows_v[j * H, pl.ds(k * _LANES, _LANES)] * wlane(0)
                       for k in range(KD)]
                for n in range(1, H):
                    p = j * H + n
                    w = wlane(n)
                    for k in range(KD):
                        acc[k] = acc[k] + rows_v[p, pl.ds(k * _LANES, _LANES)] * w

                for k in range(KD):
                    out_v[s, pl.ds(j * D + k * _LANES, _LANES)] = acc[k]

        for i in range(_NBUF - 1):
            gcopy(i, i).start()

        def round_(t, carry):
            s0 = t * _NBUF
            for b in range(_NBUF):
                s = s0 + b
                gcopy(s, b).wait()
                compute(s, rows_bufs[b])
                nxt = s + _NBUF - 1

                @pl.when(nxt < S)
                def _():
                    gcopy(nxt, (b + _NBUF - 1) % _NBUF).start()
            return carry

        lax.fori_loop(0, S // _NBUF, round_, 0)
        pltpu.sync_copy(out_v, out_hbm.at[pl.ds(wid * S, S)])

    return bag


def kernel(X, X_w, W):
    B, H = X.shape
    V, D = W.shape
    S = B // (_NW * _G)
    R = _G * H
    Wt = jnp.pad(W[:V - 1], ((0, 0), (0, _PD - D)))
    Xr = X.astype(jnp.int32).reshape(_NW, S, R)
    spread = ((jnp.arange(S)[:, None] * (_PR - R)
               + jnp.arange(_PR - R)[None, :]) * 997) % (V - 1)
    spread = jnp.broadcast_to(spread[None].astype(jnp.int32), (_NW, S, _PR - R))
    Xp = jnp.concatenate([Xr, spread], axis=2)
    Wr = X_w.astype(jnp.float32).reshape(_NW, S, R)
    Wp = jnp.pad(Wr, ((0, 0), (0, 0), (0, _PR - R)))
    out2 = _make_embedding_bag(B, H, D, V)(Wt, Xp, Wp)
    return out2.reshape(B, D)


# speedup vs baseline: 3.1982x; 1.2891x over previous
import functools

import jax
import jax.numpy as jnp
from jax import lax
from jax.experimental import pallas as pl
from jax.experimental.pallas import tpu as pltpu
from jax.experimental.pallas import tpu_sc as plsc

_INFO = plsc.get_sparse_core_info()
_NC = _INFO.num_cores
_NS = _INFO.num_subcores
_NW = _NC * _NS
_LANES = _INFO.num_lanes
_G = 2
_PR = 128
_PD = 128
_NBUF = 4


@functools.lru_cache(maxsize=None)
def _make_embedding_bag(B, H, D, V):
    assert B % (_NW * _G) == 0
    S = B // (_NW * _G)
    R = _G * H
    RF = -(-R // 8) * 8
    assert RF <= _PR
    KD = D // _LANES

    mesh = plsc.VectorSubcoreMesh(core_axis_name="c", subcore_axis_name="s")

    @functools.partial(
        pl.kernel,
        mesh=mesh,
        compiler_params=pltpu.CompilerParams(use_tc_tiling_on_sc=False),
        out_type=jax.ShapeDtypeStruct((B // _G, _G * D), jnp.float32),
        scratch_types=[
            pltpu.VMEM((S, _PR), jnp.int32),
            pltpu.VMEM((S, _PR), jnp.float32),
            [pltpu.VMEM((RF, D), jnp.float32)] * _NBUF,
            pltpu.VMEM((S, _G * D), jnp.float32),
            [pltpu.SemaphoreType.DMA] * _NBUF,
        ],
    )
    def bag(table_hbm, idx_hbm, wgt_hbm, out_hbm,
            idx_v, wgt_v, rows_bufs, out_v, sems):
        wid = lax.axis_index("s") * _NC + lax.axis_index("c")
        pltpu.sync_copy(idx_hbm.at[wid], idx_v)
        pltpu.sync_copy(wgt_hbm.at[wid], wgt_v)

        def gcopy(s, b):
            return pltpu.make_async_copy(
                table_hbm.at[idx_v.at[s, pl.ds(0, RF)]],
                rows_bufs[b], sems[b])

        def compute(s, rows_v):
            for j in range(_G):
                base = j * H
                chunk_offs = [0, 16, 32, H - _LANES]
                wvecs = [wgt_v[s, pl.ds(base + o, _LANES)] for o in chunk_offs]

                def wlane(n):
                    if n < 48:
                        return wvecs[n // 16][n % 16]
                    return wvecs[3][n - (H - _LANES)]

                acc = [rows_v[j * H, pl.ds(k * _LANES, _LANES)] * wlane(0)
                       for k in range(KD)]
                for n in range(1, H):
                    p = j * H + n
                    w = wlane(n)
                    for k in range(KD):
                        acc[k] = acc[k] + rows_v[p, pl.ds(k * _LANES, _LANES)] * w

                for k in range(KD):
                    out_v[s, pl.ds(j * D + k * _LANES, _LANES)] = acc[k]

        for i in range(_NBUF - 1):
            gcopy(i, i).start()

        def round_(t, carry):
            s0 = t * _NBUF
            for b in range(_NBUF):
                s = s0 + b
                gcopy(s, b).wait()
                compute(s, rows_bufs[b])
                nxt = s + _NBUF - 1

                @pl.when(nxt < S)
                def _():
                    gcopy(nxt, (b + _NBUF - 1) % _NBUF).start()
            return carry

        lax.fori_loop(0, S // _NBUF, round_, 0)
        pltpu.sync_copy(out_v, out_hbm.at[pl.ds(wid * S, S)])

    return bag


def kernel(X, X_w, W):
    B, H = X.shape
    V, D = W.shape
    S = B // (_NW * _G)
    R = _G * H
    Wt = jnp.pad(W[:V - 1], ((0, 0), (0, _PD - D))).reshape(2 * (V - 1), D)
    Xr = X.astype(jnp.int32).reshape(_NW, S, R) * 2
    spread = 2 * (((jnp.arange(S)[:, None] * (_PR - R)
                    + jnp.arange(_PR - R)[None, :]) * 997) % (V - 1))
    spread = jnp.broadcast_to(spread[None].astype(jnp.int32), (_NW, S, _PR - R))
    Xp = jnp.concatenate([Xr, spread], axis=2)
    Wr = X_w.astype(jnp.float32).reshape(_NW, S, R)
    Wp = jnp.pad(Wr, ((0, 0), (0, 0), (0, _PR - R)))
    out2 = _make_embedding_bag(B, H, D, V)(Wt, Xp, Wp)
    return out2.reshape(B, D)
